# baseline (speedup 1.0000x reference)
import jax
import jax.numpy as jnp
from jax.experimental import pallas as pl
from jax.experimental.pallas import tpu as pltpu


def _round_up(x, m):
    return ((x + m - 1) // m) * m


def _fused_kernel(x_ref, a1_ref, b1_ref, a2_ref, b2_ref, w3_ref, b3_ref,
                  w4_ref, b4_ref, w5_ref, b5_ref, out_ref,
                  a1c_ref, a2c_ref):
    f32, bf16 = jnp.float32, jnp.bfloat16

    def dot(a, b):
        return jnp.dot(a, b, preferred_element_type=f32)

    @pl.when(pl.program_id(0) == 0)
    def _init():
        for k in range(4):
            a1c_ref[:, 896 * k:896 * k + 864] = a1_ref[k]
            a2c_ref[:, 256 * k:256 * k + 192] = a2_ref[k]

    x = x_ref[...]

    y1 = dot(x, a1c_ref[...])
    m1 = jnp.maximum(jnp.maximum(y1[:, 0:896], y1[:, 896:1792]),
                     jnp.maximum(y1[:, 1792:2688], y1[:, 2688:3584]))
    p1 = jnp.maximum(m1[:, :864] + b1_ref[...], 0.0).astype(bf16)

    y2 = dot(p1, a2c_ref[...])
    m2 = jnp.maximum(jnp.maximum(y2[:, 0:256], y2[:, 256:512]),
                     jnp.maximum(y2[:, 512:768], y2[:, 768:1024]))
    p2 = jnp.maximum(m2[:, :192] + b2_ref[...], 0.0).astype(bf16)

    h = jnp.maximum(dot(p2, w3_ref[...]) + b3_ref[...], 0.0).astype(bf16)
    h = jnp.maximum(dot(h, w4_ref[...]) + b4_ref[...], 0.0).astype(bf16)
    out = dot(h, w5_ref[...]) + b5_ref[...]
    out_ref[...] = out[:, :10].astype(out_ref.dtype)


def _pick_batch_tile(b):
    if b >= 2048:
        return 1024
    if b >= 1024:
        return 512
    if b >= 32:
        return _round_up((b + 1) // 2, 16)
    return _round_up(b, 16)


def kernel(a1, b1, a2, b2, w3, b3, w4, b4, w5, b5, x):
    b = x.shape[0]
    x_flat = x.reshape(b, 28 * 28).astype(jnp.bfloat16)

    tb = _pick_batch_tile(b)
    bpad = _round_up(b, tb)
    if bpad != b:
        x_flat = jnp.pad(x_flat, ((0, bpad - b), (0, 0)))

    consts = [a1, b1, a2, b2, w3, b3, w4, b4, w5, b5]

    def _const_spec(arr):
        return pl.BlockSpec(arr.shape, lambda i, _z=(0,) * arr.ndim: _z)

    out = pl.pallas_call(
        _fused_kernel,
        out_shape=jax.ShapeDtypeStruct((bpad, 10), jnp.float32),
        grid=(bpad // tb,),
        in_specs=[pl.BlockSpec((tb, 28 * 28), lambda i: (i, 0))]
                 + [_const_spec(c) for c in consts],
        out_specs=pl.BlockSpec((tb, 10), lambda i: (i, 0)),
        scratch_shapes=[
            pltpu.VMEM((784, 4 * 896), jnp.bfloat16),
            pltpu.VMEM((864, 4 * 256), jnp.bfloat16),
        ],
        compiler_params=pltpu.CompilerParams(
            dimension_semantics=("arbitrary",),
            vmem_limit_bytes=64 * 1024 * 1024,
        ),
    )(x_flat, *consts)
    return out[:b]


# speedup vs baseline: 1.0073x; 1.0073x over previous
import jax
import jax.numpy as jnp
from jax.experimental import pallas as pl
from jax.experimental.pallas import tpu as pltpu


def _round_up(x, m):
    return ((x + m - 1) // m) * m


def _fused_kernel(x_ref, a1_ref, b1_ref, a2_ref, b2_ref, w3_ref, b3_ref,
                  w4_ref, b4_ref, w5_ref, b5_ref, out_ref,
                  a1c_ref, a2c_ref):
    f32, bf16 = jnp.float32, jnp.bfloat16

    def dot(a, b):
        return jnp.dot(a, b, preferred_element_type=f32)

    @pl.when(pl.program_id(0) == 0)
    def _init():
        for k in range(4):
            a1c_ref[:, 896 * k:896 * k + 864] = a1_ref[k]
            a2c_ref[:, 256 * k:256 * k + 192] = a2_ref[k]

    x = x_ref[...]

    y1 = dot(x, a1c_ref[...])
    m1 = jnp.maximum(jnp.maximum(y1[:, 0:896], y1[:, 896:1792]),
                     jnp.maximum(y1[:, 1792:2688], y1[:, 2688:3584]))
    p1 = jnp.maximum(m1[:, :864] + b1_ref[...], 0.0).astype(bf16)

    y2 = dot(p1, a2c_ref[...])
    m2 = jnp.maximum(jnp.maximum(y2[:, 0:256], y2[:, 256:512]),
                     jnp.maximum(y2[:, 512:768], y2[:, 768:1024]))
    p2 = jnp.maximum(m2[:, :192] + b2_ref[...], 0.0).astype(bf16)

    h = jnp.maximum(dot(p2, w3_ref[...]) + b3_ref[...], 0.0).astype(bf16)
    h = jnp.maximum(dot(h, w4_ref[...]) + b4_ref[...], 0.0).astype(bf16)
    out = dot(h, w5_ref[...]) + b5_ref[...]
    out_ref[...] = out[:, :10].astype(out_ref.dtype)


def _pick_batch_tile(b):
    if b >= 4096:
        return 2048
    if b >= 2048:
        return 1024
    if b >= 1024:
        return 512
    if b >= 32:
        return _round_up((b + 1) // 2, 16)
    return _round_up(b, 16)


def kernel(a1, b1, a2, b2, w3, b3, w4, b4, w5, b5, x):
    b = x.shape[0]
    x_flat = x.reshape(b, 28 * 28).astype(jnp.bfloat16)

    tb = _pick_batch_tile(b)
    bpad = _round_up(b, tb)
    if bpad != b:
        x_flat = jnp.pad(x_flat, ((0, bpad - b), (0, 0)))

    consts = [a1, b1, a2, b2, w3, b3, w4, b4, w5, b5]

    def _const_spec(arr):
        return pl.BlockSpec(arr.shape, lambda i, _z=(0,) * arr.ndim: _z)

    out = pl.pallas_call(
        _fused_kernel,
        out_shape=jax.ShapeDtypeStruct((bpad, 10), jnp.float32),
        grid=(bpad // tb,),
        in_specs=[pl.BlockSpec((tb, 28 * 28), lambda i: (i, 0))]
                 + [_const_spec(c) for c in consts],
        out_specs=pl.BlockSpec((tb, 10), lambda i: (i, 0)),
        scratch_shapes=[
            pltpu.VMEM((784, 4 * 896), jnp.bfloat16),
            pltpu.VMEM((864, 4 * 256), jnp.bfloat16),
        ],
        compiler_params=pltpu.CompilerParams(
            dimension_semantics=("arbitrary",),
            vmem_limit_bytes=64 * 1024 * 1024,
        ),
    )(x_flat, *consts)
    return out[:b]
